# Initial kernel scaffold; baseline (speedup 1.0000x reference)
import jax
import jax.numpy as jnp
from jax.experimental import pallas as pl


def kernel(image, feature_map, gt_boxes):
    raise NotImplementedError("write your pallas kernel here")


# speedup vs baseline: 3.5452x; 3.5452x over previous
import numpy as np
import jax
import jax.numpy as jnp
from jax import lax
from jax.experimental import pallas as pl
from jax.experimental.pallas import tpu as pltpu

_IM_H, _IM_W = 512, 512
_FEAT_H, _FEAT_W = 32, 32
_STRIDE = 16
_POS_THRES, _NEG_THRES = 0.7, 0.3
_N_SAMPLES = 256
_N_POS = _N_SAMPLES // 2
_A = _FEAT_H * _FEAT_W * 9
_R, _C = 72, 128
_INF_BITS = np.int32(0x7F800000)


def _make_anchors():
    ratios = [0.5, 1.0, 2.0]
    scales = [8.0, 16.0, 32.0]
    hs, ws = [], []
    for r in ratios:
        for s in scales:
            hs.append(_STRIDE * s * np.sqrt(r))
            ws.append(_STRIDE * s * np.sqrt(1.0 / r))
    hs = np.array(hs, dtype=np.float64)
    ws = np.array(ws, dtype=np.float64)
    sy = (np.arange(_FEAT_H) + 0.5) * _STRIDE
    sx = (np.arange(_FEAT_W) + 0.5) * _STRIDE
    cy, cx = np.meshgrid(sy, sx, indexing="ij")
    cy = cy.reshape(-1, 1)
    cx = cx.reshape(-1, 1)
    anchors = np.stack(
        [cy - 0.5 * hs, cx - 0.5 * ws, cy + 0.5 * hs, cx + 0.5 * ws], axis=-1
    )
    return anchors.reshape(-1, 4).astype(np.float32)


_ANCHORS = _make_anchors()
_AY1 = _ANCHORS[:, 0].reshape(_R, _C)
_AX1 = _ANCHORS[:, 1].reshape(_R, _C)
_AY2 = _ANCHORS[:, 2].reshape(_R, _C)
_AX2 = _ANCHORS[:, 3].reshape(_R, _C)
_INSIDE = (
    (_ANCHORS[:, 0] >= 0)
    & (_ANCHORS[:, 1] >= 0)
    & (_ANCHORS[:, 2] <= _IM_H)
    & (_ANCHORS[:, 3] <= _IM_W)
).reshape(_R, _C)
_AREA_A = ((_ANCHORS[:, 2] - _ANCHORS[:, 0]) * (_ANCHORS[:, 3] - _ANCHORS[:, 1])
           ).reshape(_R, _C)
_SKEY = jax.random.key(42)
_RND_P = np.asarray(jax.random.uniform(_SKEY, (_A,))).reshape(_R, _C)
_RND_N = np.asarray(
    jax.random.uniform(jax.random.fold_in(_SKEY, 1), (_A,))
).reshape(_R, _C)
_RND_P_BITS = _RND_P.view(np.int32)
_RND_N_BITS = _RND_N.view(np.int32)


def _body(gt_ref, ay1_ref, ax1_ref, ay2_ref, ax2_ref, area_ref, inside_ref,
          rndp_ref, rndn_ref,
          dy_ref, dx_ref, dh_ref, dw_ref, lab_ref):
    ay1 = ay1_ref[...]
    ax1 = ax1_ref[...]
    ay2 = ay2_ref[...]
    ax2 = ax2_ref[...]
    area_a = area_ref[...]
    inside = inside_ref[...] != 0

    neg_two = jnp.full((_R, _C), -2.0, dtype=jnp.float32)
    zero = jnp.zeros((_R, _C), dtype=jnp.float32)
    zero_i = jnp.zeros((_R, _C), dtype=jnp.int32)

    def gt_step(j, carry):
        curmax, by1, bx1, by2, bx2, eq = carry
        gy1 = gt_ref[j, 0]
        gx1 = gt_ref[j, 1]
        gy2 = gt_ref[j, 2]
        gx2 = gt_ref[j, 3]
        tl_y = jnp.maximum(ay1, gy1)
        tl_x = jnp.maximum(ax1, gx1)
        br_y = jnp.minimum(ay2, gy2)
        br_x = jnp.minimum(ax2, gx2)
        h = jnp.maximum(br_y - tl_y, 0.0)
        w = jnp.maximum(br_x - tl_x, 0.0)
        inter = h * w
        area_g = (gy2 - gy1) * (gx2 - gx1)
        iou = inter / ((area_a + area_g) - inter)
        iou_m = jnp.where(inside, iou, -1.0)
        gmax = jnp.max(iou_m)
        eq = jnp.where(iou_m == gmax, 1, eq)
        upd = iou_m > curmax
        curmax = jnp.where(upd, iou_m, curmax)
        by1 = jnp.where(upd, gy1, by1)
        bx1 = jnp.where(upd, gx1, bx1)
        by2 = jnp.where(upd, gy2, by2)
        bx2 = jnp.where(upd, gx2, bx2)
        return curmax, by1, bx1, by2, bx2, eq

    curmax, by1, bx1, by2, bx2, eq = lax.fori_loop(
        0, 100, gt_step, (neg_two, zero, zero, zero, zero, zero_i))

    labels = jnp.full((_R, _C), -1, dtype=jnp.int32)
    labels = jnp.where(inside & (curmax < _NEG_THRES), 0, labels)
    labels = jnp.where(inside & (eq != 0), 1, labels)
    labels = jnp.where(inside & (curmax >= _POS_THRES), 1, labels)

    pos_mask = labels == 1
    neg_mask = labels == 0
    pos_count = jnp.sum(pos_mask.astype(jnp.int32))
    neg_count = jnp.sum(neg_mask.astype(jnp.int32))
    bits_p = jnp.where(pos_mask, rndp_ref[...], _INF_BITS)
    bits_n = jnp.where(neg_mask, rndn_ref[...], _INF_BITS)
    n_neg = jnp.where(pos_count < _N_POS, pos_count, _N_POS)
    k_p = jnp.int32(_N_POS)
    k_n = jnp.maximum(n_neg, 1)

    def bs_step(_, c):
        lo_p, hi_p, lo_n, hi_n = c
        mid_p = lo_p + (hi_p - lo_p) // 2
        mid_n = lo_n + (hi_n - lo_n) // 2
        cnt_p = jnp.sum((bits_p <= mid_p).astype(jnp.int32))
        cnt_n = jnp.sum((bits_n <= mid_n).astype(jnp.int32))
        ge_p = cnt_p >= k_p
        ge_n = cnt_n >= k_n
        hi_p = jnp.where(ge_p, mid_p, hi_p)
        lo_p = jnp.where(ge_p, lo_p, mid_p + 1)
        hi_n = jnp.where(ge_n, mid_n, hi_n)
        lo_n = jnp.where(ge_n, lo_n, mid_n + 1)
        return lo_p, hi_p, lo_n, hi_n

    z = jnp.int32(0)
    lo_p, _, lo_n, _ = lax.fori_loop(
        0, 31, bs_step, (z, _INF_BITS + 0, z, _INF_BITS + 0))

    kill_p = (pos_count > _N_POS) & pos_mask & (bits_p > lo_p)
    labels = jnp.where(kill_p, -1, labels)
    over_neg = neg_count > n_neg
    kill_n = over_neg & (
        ((n_neg > 0) & neg_mask & (bits_n > lo_n)) | ((n_neg == 0) & neg_mask)
    )
    labels = jnp.where(kill_n, -1, labels)
    lab_ref[...] = labels

    bh = by2 - by1
    bw = bx2 - bx1
    bcy = by1 + 0.5 * bh
    bcx = bx1 + 0.5 * bw
    eps = jnp.float32(np.finfo(np.float32).eps)
    ah = jnp.maximum(ay2 - ay1, eps)
    aw = jnp.maximum(ax2 - ax1, eps)
    acy = ay1 + 0.5 * (ay2 - ay1)
    acx = ax1 + 0.5 * (ax2 - ax1)
    dy = (bcy - acy) / ah
    dx = (bcx - acx) / aw
    dh = jnp.log(bh / ah)
    dw = jnp.log(bw / aw)
    dy_ref[...] = jnp.where(inside, dy, 0.0)
    dx_ref[...] = jnp.where(inside, dx, 0.0)
    dh_ref[...] = jnp.where(inside, dh, 0.0)
    dw_ref[...] = jnp.where(inside, dw, 0.0)


def kernel(image, feature_map, gt_boxes):
    del image, feature_map
    plane = jax.ShapeDtypeStruct((_R, _C), jnp.float32)
    outs = pl.pallas_call(
        _body,
        out_shape=(
            plane, plane, plane, plane,
            jax.ShapeDtypeStruct((_R, _C), jnp.int32),
        ),
        in_specs=[
            pl.BlockSpec(memory_space=pltpu.SMEM),
            pl.BlockSpec(memory_space=pltpu.VMEM),
            pl.BlockSpec(memory_space=pltpu.VMEM),
            pl.BlockSpec(memory_space=pltpu.VMEM),
            pl.BlockSpec(memory_space=pltpu.VMEM),
            pl.BlockSpec(memory_space=pltpu.VMEM),
            pl.BlockSpec(memory_space=pltpu.VMEM),
            pl.BlockSpec(memory_space=pltpu.VMEM),
            pl.BlockSpec(memory_space=pltpu.VMEM),
        ],
    )(
        gt_boxes,
        jnp.asarray(_AY1), jnp.asarray(_AX1),
        jnp.asarray(_AY2), jnp.asarray(_AX2),
        jnp.asarray(_AREA_A),
        jnp.asarray(_INSIDE.astype(np.int32)),
        jnp.asarray(_RND_P_BITS), jnp.asarray(_RND_N_BITS),
    )
    dy, dx, dh, dw, labels = outs
    locs = jnp.stack(
        [dy.reshape(_A), dx.reshape(_A), dh.reshape(_A), dw.reshape(_A)],
        axis=1,
    )
    return locs, labels.reshape(_A), jnp.asarray(_ANCHORS)


# speedup vs baseline: 4.4916x; 1.2670x over previous
import numpy as np
import jax
import jax.numpy as jnp
from jax import lax
from jax.experimental import pallas as pl
from jax.experimental.pallas import tpu as pltpu

_IM_H, _IM_W = 512, 512
_FEAT_H, _FEAT_W = 32, 32
_STRIDE = 16
_POS_THRES, _NEG_THRES = 0.7, 0.3
_N_SAMPLES = 256
_N_POS = _N_SAMPLES // 2
_A = _FEAT_H * _FEAT_W * 9
_R, _C = 72, 128
_INF_BITS = np.int32(0x7F800000)


def _make_anchors():
    ratios = [0.5, 1.0, 2.0]
    scales = [8.0, 16.0, 32.0]
    hs, ws = [], []
    for r in ratios:
        for s in scales:
            hs.append(_STRIDE * s * np.sqrt(r))
            ws.append(_STRIDE * s * np.sqrt(1.0 / r))
    hs = np.array(hs, dtype=np.float64)
    ws = np.array(ws, dtype=np.float64)
    sy = (np.arange(_FEAT_H) + 0.5) * _STRIDE
    sx = (np.arange(_FEAT_W) + 0.5) * _STRIDE
    cy, cx = np.meshgrid(sy, sx, indexing="ij")
    cy = cy.reshape(-1, 1)
    cx = cx.reshape(-1, 1)
    anchors = np.stack(
        [cy - 0.5 * hs, cx - 0.5 * ws, cy + 0.5 * hs, cx + 0.5 * ws], axis=-1
    )
    return anchors.reshape(-1, 4).astype(np.float32)


_ANCHORS = _make_anchors()
_AY1 = _ANCHORS[:, 0].reshape(_R, _C)
_AX1 = _ANCHORS[:, 1].reshape(_R, _C)
_AY2 = _ANCHORS[:, 2].reshape(_R, _C)
_AX2 = _ANCHORS[:, 3].reshape(_R, _C)
_INSIDE = (
    (_ANCHORS[:, 0] >= 0)
    & (_ANCHORS[:, 1] >= 0)
    & (_ANCHORS[:, 2] <= _IM_H)
    & (_ANCHORS[:, 3] <= _IM_W)
).reshape(_R, _C)
_AREA_A = ((_ANCHORS[:, 2] - _ANCHORS[:, 0]) * (_ANCHORS[:, 3] - _ANCHORS[:, 1])
           ).reshape(_R, _C)
_TF_ROT = ((13, 15, 26, 6), (17, 29, 16, 24))


def _tf_pair(k1, k2, x0, x1):
    ks = (np.uint32(k1), np.uint32(k2),
          np.uint32(np.uint32(k1) ^ np.uint32(k2) ^ np.uint32(0x1BD11BDA)))
    x0 = (x0 + ks[0]).astype(np.uint32)
    x1 = (x1 + ks[1]).astype(np.uint32)
    for r in range(5):
        for d in _TF_ROT[r % 2]:
            x0 = (x0 + x1).astype(np.uint32)
            x1 = ((x1 << np.uint32(d)) | (x1 >> np.uint32(32 - d))).astype(np.uint32)
            x1 = (x0 ^ x1).astype(np.uint32)
        x0 = (x0 + ks[(r + 1) % 3]).astype(np.uint32)
        x1 = (x1 + ks[(r + 2) % 3] + np.uint32(r + 1)).astype(np.uint32)
    return x0, x1


def _tf_uniform(k, n):
    b1, b2 = _tf_pair(k[0], k[1], np.zeros(n, np.uint32),
                      np.arange(n, dtype=np.uint32))
    bits = (b1 ^ b2).astype(np.uint32)
    f = ((bits >> np.uint32(9)) | np.uint32(0x3F800000)).view(np.float32)
    return np.maximum(np.float32(0.0), f - np.float32(1.0))


def _tf_fold_in(k, data):
    b0, b1 = _tf_pair(k[0], k[1],
                      np.array([(data >> 32) & 0xFFFFFFFF], np.uint32),
                      np.array([data & 0xFFFFFFFF], np.uint32))
    return b0[0], b1[0]


_SKEY = (np.uint32(0), np.uint32(42))
_RND_P = _tf_uniform(_SKEY, _A).reshape(_R, _C)
_RND_N = _tf_uniform(_tf_fold_in(_SKEY, 1), _A).reshape(_R, _C)
_RND_P_BITS = _RND_P.view(np.int32)
_RND_N_BITS = _RND_N.view(np.int32)


def _body(gt_ref, ay1_ref, ax1_ref, ay2_ref, ax2_ref, area_ref, inside_ref,
          rndp_ref, rndn_ref,
          dy_ref, dx_ref, dh_ref, dw_ref, lab_ref):
    inside = inside_ref[...] != 0

    neg_two = jnp.full((_R, _C), -2.0, dtype=jnp.float32)
    zero = jnp.zeros((_R, _C), dtype=jnp.float32)
    zero_i = jnp.zeros((_R, _C), dtype=jnp.int32)

    def one_gt(j, curmax, bestj, eq):
        gy1 = gt_ref[j, 0]
        gx1 = gt_ref[j, 1]
        gy2 = gt_ref[j, 2]
        gx2 = gt_ref[j, 3]
        tl_y = jnp.maximum(ay1_ref[...], gy1)
        tl_x = jnp.maximum(ax1_ref[...], gx1)
        br_y = jnp.minimum(ay2_ref[...], gy2)
        br_x = jnp.minimum(ax2_ref[...], gx2)
        h = jnp.maximum(br_y - tl_y, 0.0)
        w = jnp.maximum(br_x - tl_x, 0.0)
        inter = h * w
        area_g = (gy2 - gy1) * (gx2 - gx1)
        iou = inter / ((area_ref[...] + area_g) - inter)
        iou_m = jnp.where(inside_ref[...] != 0, iou, -1.0)
        gmax = jnp.max(iou_m)
        eq = jnp.where(iou_m == gmax, 1, eq)
        upd = iou_m > curmax
        curmax = jnp.where(upd, iou_m, curmax)
        bestj = jnp.where(upd, j, bestj)
        return curmax, bestj, eq

    def gt_step(jj, carry):
        curmax, bestj, eq = carry
        j0 = jj * 2
        curmax, bestj, eq = one_gt(j0, curmax, bestj, eq)
        curmax, bestj, eq = one_gt(j0 + 1, curmax, bestj, eq)
        return curmax, bestj, eq

    curmax, bestj, eq = lax.fori_loop(
        0, 50, gt_step, (neg_two, zero_i, zero_i))

    def exp_step(jj, carry):
        by1, bx1, by2, bx2 = carry
        for j in (jj * 2, jj * 2 + 1):
            sel = bestj == j
            by1 = jnp.where(sel, gt_ref[j, 0], by1)
            bx1 = jnp.where(sel, gt_ref[j, 1], bx1)
            by2 = jnp.where(sel, gt_ref[j, 2], by2)
            bx2 = jnp.where(sel, gt_ref[j, 3], bx2)
        return by1, bx1, by2, bx2

    by1, bx1, by2, bx2 = lax.fori_loop(
        0, 50, exp_step, (zero, zero, zero, zero))

    labels = jnp.full((_R, _C), -1, dtype=jnp.int32)
    labels = jnp.where(inside & (curmax < _NEG_THRES), 0, labels)
    labels = jnp.where(inside & (eq != 0), 1, labels)
    labels = jnp.where(inside & (curmax >= _POS_THRES), 1, labels)

    pos_mask = labels == 1
    neg_mask = labels == 0
    pos_count = jnp.sum(pos_mask.astype(jnp.int32))
    neg_count = jnp.sum(neg_mask.astype(jnp.int32))
    bits_p = jnp.where(pos_mask, rndp_ref[...], _INF_BITS)
    bits_n = jnp.where(neg_mask, rndn_ref[...], _INF_BITS)
    n_neg = jnp.where(pos_count < _N_POS, pos_count, _N_POS)
    k_p = jnp.int32(_N_POS)
    k_n = jnp.maximum(n_neg, 1)

    def bs_step(_, c):
        lo_p, hi_p, lo_n, hi_n = c
        mid_p = lo_p + (hi_p - lo_p) // 2
        mid_n = lo_n + (hi_n - lo_n) // 2
        cnt_p = jnp.sum((bits_p <= mid_p).astype(jnp.int32))
        cnt_n = jnp.sum((bits_n <= mid_n).astype(jnp.int32))
        ge_p = cnt_p >= k_p
        ge_n = cnt_n >= k_n
        hi_p = jnp.where(ge_p, mid_p, hi_p)
        lo_p = jnp.where(ge_p, lo_p, mid_p + 1)
        hi_n = jnp.where(ge_n, mid_n, hi_n)
        lo_n = jnp.where(ge_n, lo_n, mid_n + 1)
        return lo_p, hi_p, lo_n, hi_n

    z = jnp.int32(0)
    lo_p, _, lo_n, _ = lax.fori_loop(
        0, 31, bs_step, (z, _INF_BITS + 0, z, _INF_BITS + 0))

    kill_p = (pos_count > _N_POS) & pos_mask & (bits_p > lo_p)
    labels = jnp.where(kill_p, -1, labels)
    over_neg = neg_count > n_neg
    kill_n = over_neg & (
        ((n_neg > 0) & neg_mask & (bits_n > lo_n)) | ((n_neg == 0) & neg_mask)
    )
    labels = jnp.where(kill_n, -1, labels)
    lab_ref[...] = labels

    bh = by2 - by1
    bw = bx2 - bx1
    bcy = by1 + 0.5 * bh
    bcx = bx1 + 0.5 * bw
    eps = jnp.float32(np.finfo(np.float32).eps)
    ay1 = ay1_ref[...]
    ax1 = ax1_ref[...]
    ay2 = ay2_ref[...]
    ax2 = ax2_ref[...]
    ah = jnp.maximum(ay2 - ay1, eps)
    aw = jnp.maximum(ax2 - ax1, eps)
    acy = ay1 + 0.5 * (ay2 - ay1)
    acx = ax1 + 0.5 * (ax2 - ax1)
    dy = (bcy - acy) / ah
    dx = (bcx - acx) / aw
    dh = jnp.log(bh / ah)
    dw = jnp.log(bw / aw)
    dy_ref[...] = jnp.where(inside, dy, 0.0)
    dx_ref[...] = jnp.where(inside, dx, 0.0)
    dh_ref[...] = jnp.where(inside, dh, 0.0)
    dw_ref[...] = jnp.where(inside, dw, 0.0)


def kernel(image, feature_map, gt_boxes):
    del image, feature_map
    plane = jax.ShapeDtypeStruct((_R, _C), jnp.float32)
    outs = pl.pallas_call(
        _body,
        out_shape=(
            plane, plane, plane, plane,
            jax.ShapeDtypeStruct((_R, _C), jnp.int32),
        ),
        in_specs=[
            pl.BlockSpec(memory_space=pltpu.SMEM),
            pl.BlockSpec(memory_space=pltpu.VMEM),
            pl.BlockSpec(memory_space=pltpu.VMEM),
            pl.BlockSpec(memory_space=pltpu.VMEM),
            pl.BlockSpec(memory_space=pltpu.VMEM),
            pl.BlockSpec(memory_space=pltpu.VMEM),
            pl.BlockSpec(memory_space=pltpu.VMEM),
            pl.BlockSpec(memory_space=pltpu.VMEM),
            pl.BlockSpec(memory_space=pltpu.VMEM),
        ],
    )(
        gt_boxes,
        jnp.asarray(_AY1), jnp.asarray(_AX1),
        jnp.asarray(_AY2), jnp.asarray(_AX2),
        jnp.asarray(_AREA_A),
        jnp.asarray(_INSIDE.astype(np.int32)),
        jnp.asarray(_RND_P_BITS), jnp.asarray(_RND_N_BITS),
    )
    dy, dx, dh, dw, labels = outs
    locs = jnp.stack(
        [dy.reshape(_A), dx.reshape(_A), dh.reshape(_A), dw.reshape(_A)],
        axis=1,
    )
    return locs, labels.reshape(_A), jnp.asarray(_ANCHORS)


# speedup vs baseline: 5.6312x; 1.2537x over previous
import numpy as np
import jax
import jax.numpy as jnp
from jax import lax
from jax.experimental import pallas as pl
from jax.experimental.pallas import tpu as pltpu

_IM_H, _IM_W = 512, 512
_FEAT_H, _FEAT_W = 32, 32
_STRIDE = 16
_POS_THRES, _NEG_THRES = 0.7, 0.3
_N_SAMPLES = 256
_N_POS = _N_SAMPLES // 2
_A = _FEAT_H * _FEAT_W * 9
_R, _C = 72, 128
_INF_BITS = np.int32(0x7F800000)


def _make_anchors():
    ratios = [0.5, 1.0, 2.0]
    scales = [8.0, 16.0, 32.0]
    hs, ws = [], []
    for r in ratios:
        for s in scales:
            hs.append(_STRIDE * s * np.sqrt(r))
            ws.append(_STRIDE * s * np.sqrt(1.0 / r))
    hs = np.array(hs, dtype=np.float64)
    ws = np.array(ws, dtype=np.float64)
    sy = (np.arange(_FEAT_H) + 0.5) * _STRIDE
    sx = (np.arange(_FEAT_W) + 0.5) * _STRIDE
    cy, cx = np.meshgrid(sy, sx, indexing="ij")
    cy = cy.reshape(-1, 1)
    cx = cx.reshape(-1, 1)
    anchors = np.stack(
        [cy - 0.5 * hs, cx - 0.5 * ws, cy + 0.5 * hs, cx + 0.5 * ws], axis=-1
    )
    return anchors.reshape(-1, 4).astype(np.float32)


_ANCHORS = _make_anchors()
_AY1 = _ANCHORS[:, 0].reshape(_R, _C)
_AX1 = _ANCHORS[:, 1].reshape(_R, _C)
_AY2 = _ANCHORS[:, 2].reshape(_R, _C)
_AX2 = _ANCHORS[:, 3].reshape(_R, _C)
_INSIDE = (
    (_ANCHORS[:, 0] >= 0)
    & (_ANCHORS[:, 1] >= 0)
    & (_ANCHORS[:, 2] <= _IM_H)
    & (_ANCHORS[:, 3] <= _IM_W)
).reshape(_R, _C)
_AREA_A = ((_ANCHORS[:, 2] - _ANCHORS[:, 0]) * (_ANCHORS[:, 3] - _ANCHORS[:, 1])
           ).reshape(_R, _C)
_TF_ROT = ((13, 15, 26, 6), (17, 29, 16, 24))


def _tf_pair(k1, k2, x0, x1):
    ks = (np.uint32(k1), np.uint32(k2),
          np.uint32(np.uint32(k1) ^ np.uint32(k2) ^ np.uint32(0x1BD11BDA)))
    x0 = (x0 + ks[0]).astype(np.uint32)
    x1 = (x1 + ks[1]).astype(np.uint32)
    for r in range(5):
        for d in _TF_ROT[r % 2]:
            x0 = (x0 + x1).astype(np.uint32)
            x1 = ((x1 << np.uint32(d)) | (x1 >> np.uint32(32 - d))).astype(np.uint32)
            x1 = (x0 ^ x1).astype(np.uint32)
        x0 = (x0 + ks[(r + 1) % 3]).astype(np.uint32)
        x1 = (x1 + ks[(r + 2) % 3] + np.uint32(r + 1)).astype(np.uint32)
    return x0, x1


def _tf_uniform(k, n):
    b1, b2 = _tf_pair(k[0], k[1], np.zeros(n, np.uint32),
                      np.arange(n, dtype=np.uint32))
    bits = (b1 ^ b2).astype(np.uint32)
    f = ((bits >> np.uint32(9)) | np.uint32(0x3F800000)).view(np.float32)
    return np.maximum(np.float32(0.0), f - np.float32(1.0))


def _tf_fold_in(k, data):
    b0, b1 = _tf_pair(k[0], k[1],
                      np.array([(data >> 32) & 0xFFFFFFFF], np.uint32),
                      np.array([data & 0xFFFFFFFF], np.uint32))
    return b0[0], b1[0]


_SKEY = (np.uint32(0), np.uint32(42))
_RND_P = _tf_uniform(_SKEY, _A).reshape(_R, _C)
_RND_N = _tf_uniform(_tf_fold_in(_SKEY, 1), _A).reshape(_R, _C)
_M_P = np.round(_RND_P * np.float32(2.0 ** 23)).astype(np.int32)
_M_N = np.round(_RND_N * np.float32(2.0 ** 23)).astype(np.int32)
assert np.array_equal(_M_P.astype(np.float32) * np.float32(2.0 ** -23), _RND_P)
assert np.array_equal(_M_N.astype(np.float32) * np.float32(2.0 ** -23), _RND_N)
_M_TOP = np.int32(1 << 23)
_M_BIG = np.int32(1 << 24)


def _body(gt_ref, ay1_ref, ax1_ref, ay2_ref, ax2_ref, area_ref, inside_ref,
          rndp_ref, rndn_ref,
          dy_ref, dx_ref, dh_ref, dw_ref, lab_ref):
    inside = inside_ref[...] != 0

    neg_two = jnp.full((_R, _C), -2.0, dtype=jnp.float32)
    zero = jnp.zeros((_R, _C), dtype=jnp.float32)
    zero_i = jnp.zeros((_R, _C), dtype=jnp.int32)

    def one_gt(j, curmax, bestj, eq):
        gy1 = gt_ref[j, 0]
        gx1 = gt_ref[j, 1]
        gy2 = gt_ref[j, 2]
        gx2 = gt_ref[j, 3]
        tl_y = jnp.maximum(ay1_ref[...], gy1)
        tl_x = jnp.maximum(ax1_ref[...], gx1)
        br_y = jnp.minimum(ay2_ref[...], gy2)
        br_x = jnp.minimum(ax2_ref[...], gx2)
        h = jnp.maximum(br_y - tl_y, 0.0)
        w = jnp.maximum(br_x - tl_x, 0.0)
        inter = h * w
        area_g = (gy2 - gy1) * (gx2 - gx1)
        iou = inter / ((area_ref[...] + area_g) - inter)
        iou_m = jnp.where(inside_ref[...] != 0, iou, -1.0)
        gmax = jnp.max(iou_m)
        eq = jnp.where(iou_m == gmax, 1, eq)
        upd = iou_m > curmax
        curmax = jnp.where(upd, iou_m, curmax)
        bestj = jnp.where(upd, j, bestj)
        return curmax, bestj, eq

    def gt_step(jj, carry):
        curmax, bestj, eq = carry
        j0 = jj * 4
        for t in range(4):
            curmax, bestj, eq = one_gt(j0 + t, curmax, bestj, eq)
        return curmax, bestj, eq

    curmax, bestj, eq = lax.fori_loop(
        0, 25, gt_step, (neg_two, zero_i, zero_i))

    def exp_step(jj, carry):
        by1, bx1, by2, bx2 = carry
        for t in range(4):
            j = jj * 4 + t
            sel = bestj == j
            by1 = jnp.where(sel, gt_ref[j, 0], by1)
            bx1 = jnp.where(sel, gt_ref[j, 1], bx1)
            by2 = jnp.where(sel, gt_ref[j, 2], by2)
            bx2 = jnp.where(sel, gt_ref[j, 3], bx2)
        return by1, bx1, by2, bx2

    by1, bx1, by2, bx2 = lax.fori_loop(
        0, 25, exp_step, (zero, zero, zero, zero))

    bh = by2 - by1
    bw = bx2 - bx1
    bcy = by1 + 0.5 * bh
    bcx = bx1 + 0.5 * bw
    eps = jnp.float32(np.finfo(np.float32).eps)
    ay1 = ay1_ref[...]
    ax1 = ax1_ref[...]
    ay2 = ay2_ref[...]
    ax2 = ax2_ref[...]
    ah = jnp.maximum(ay2 - ay1, eps)
    aw = jnp.maximum(ax2 - ax1, eps)
    acy = ay1 + 0.5 * (ay2 - ay1)
    acx = ax1 + 0.5 * (ax2 - ax1)
    dy = (bcy - acy) / ah
    dx = (bcx - acx) / aw
    dh = jnp.log(bh / ah)
    dw = jnp.log(bw / aw)
    dy_ref[...] = jnp.where(inside, dy, 0.0)
    dx_ref[...] = jnp.where(inside, dx, 0.0)
    dh_ref[...] = jnp.where(inside, dh, 0.0)
    dw_ref[...] = jnp.where(inside, dw, 0.0)

    labels = jnp.full((_R, _C), -1, dtype=jnp.int32)
    labels = jnp.where(inside & (curmax < _NEG_THRES), 0, labels)
    labels = jnp.where(inside & (eq != 0), 1, labels)
    labels = jnp.where(inside & (curmax >= _POS_THRES), 1, labels)

    pos_mask = labels == 1
    neg_mask = labels == 0
    pos_count = jnp.sum(pos_mask.astype(jnp.int32))
    neg_count = jnp.sum(neg_mask.astype(jnp.int32))
    m_p = jnp.where(pos_mask, rndp_ref[...], _M_BIG)
    m_n = jnp.where(neg_mask, rndn_ref[...], _M_BIG)
    n_neg = jnp.where(pos_count < _N_POS, pos_count, _N_POS)
    k_p = jnp.int32(_N_POS)
    k_n = jnp.maximum(n_neg, 1)

    def bs_step(_, c):
        lo_p, hi_p, lo_n, hi_n = c
        mid_p = lo_p + (hi_p - lo_p) // 2
        mid_n = lo_n + (hi_n - lo_n) // 2
        cnt_p = jnp.sum((m_p <= mid_p).astype(jnp.int32))
        cnt_n = jnp.sum((m_n <= mid_n).astype(jnp.int32))
        ge_p = cnt_p >= k_p
        ge_n = cnt_n >= k_n
        hi_p = jnp.where(ge_p, mid_p, hi_p)
        lo_p = jnp.where(ge_p, lo_p, mid_p + 1)
        hi_n = jnp.where(ge_n, mid_n, hi_n)
        lo_n = jnp.where(ge_n, lo_n, mid_n + 1)
        return lo_p, hi_p, lo_n, hi_n

    z = jnp.int32(0)
    lo_p, _, lo_n, _ = lax.fori_loop(
        0, 24, bs_step, (z, _M_TOP + 0, z, _M_TOP + 0))

    kill_p = (pos_count > _N_POS) & pos_mask & (m_p > lo_p)
    labels = jnp.where(kill_p, -1, labels)
    over_neg = neg_count > n_neg
    kill_n = over_neg & (
        ((n_neg > 0) & neg_mask & (m_n > lo_n)) | ((n_neg == 0) & neg_mask)
    )
    labels = jnp.where(kill_n, -1, labels)
    lab_ref[...] = labels


def kernel(image, feature_map, gt_boxes):
    del image, feature_map
    plane = jax.ShapeDtypeStruct((_R, _C), jnp.float32)
    outs = pl.pallas_call(
        _body,
        out_shape=(
            plane, plane, plane, plane,
            jax.ShapeDtypeStruct((_R, _C), jnp.int32),
        ),
        in_specs=[
            pl.BlockSpec(memory_space=pltpu.SMEM),
            pl.BlockSpec(memory_space=pltpu.VMEM),
            pl.BlockSpec(memory_space=pltpu.VMEM),
            pl.BlockSpec(memory_space=pltpu.VMEM),
            pl.BlockSpec(memory_space=pltpu.VMEM),
            pl.BlockSpec(memory_space=pltpu.VMEM),
            pl.BlockSpec(memory_space=pltpu.VMEM),
            pl.BlockSpec(memory_space=pltpu.VMEM),
            pl.BlockSpec(memory_space=pltpu.VMEM),
        ],
    )(
        gt_boxes,
        jnp.asarray(_AY1), jnp.asarray(_AX1),
        jnp.asarray(_AY2), jnp.asarray(_AX2),
        jnp.asarray(_AREA_A),
        jnp.asarray(_INSIDE.astype(np.int32)),
        jnp.asarray(_M_P), jnp.asarray(_M_N),
    )
    dy, dx, dh, dw, labels = outs
    locs = jnp.stack(
        [dy.reshape(_A), dx.reshape(_A), dh.reshape(_A), dw.reshape(_A)],
        axis=1,
    )
    return locs, labels.reshape(_A), jnp.asarray(_ANCHORS)
